# baseline (speedup 1.0000x reference)
import functools

import jax
import jax.numpy as jnp
from jax import lax
from jax.experimental import pallas as pl
from jax.experimental.pallas import tpu as pltpu
from jax.experimental.pallas import tpu_sc as plsc

_F32 = jnp.float32
_LANES = 384
_CH = 1
_NTILES = 16
_ZROWS = 1000


def _leaky(x):
    return jnp.where(x >= 0, x, 0.5 * x)


@functools.lru_cache(maxsize=None)
def _make_sc_agg(n, nz, g, stage=False):
    gpt = g // _NTILES
    npair = gpt // (2 * _CH)
    mesh = plsc.VectorSubcoreMesh(core_axis_name="c", subcore_axis_name="s")

    @functools.partial(
        pl.kernel,
        mesh=mesh,
        compiler_params=pltpu.CompilerParams(use_tc_tiling_on_sc=False),
        out_type=[jax.ShapeDtypeStruct((n, 32), _F32),
                  jax.ShapeDtypeStruct((n, 32), _F32)],
        scratch_types=[
            pltpu.VMEM((_CH, 2, _LANES), jnp.int32),
            pltpu.VMEM((_CH, 2, _LANES), jnp.int32),
            pltpu.VMEM((_CH * _LANES, 32), _F32),
            pltpu.VMEM((_CH * _LANES, 32), _F32),
            pltpu.VMEM_SHARED((nz, 32), _F32),
        ] + ([pltpu.VMEM_SHARED((n, 32), _F32)] if stage else []) + [
            pltpu.SemaphoreType.DMA,
            pltpu.SemaphoreType.DMA,
            pltpu.SemaphoreType.DMA,
            pltpu.SemaphoreType.DMA,
        ],
    )
    def agg(nf0, nf1, idx2, zrows, out0, out1, *refs):
        if stage:
            idxa, idxb, bufa, bufb, acc, table, semga, semgb, semsa, semsb = refs
        else:
            idxa, idxb, bufa, bufb, acc, semga, semgb, semsa, semsb = refs
            table = None
        c = lax.axis_index("c")
        s = lax.axis_index("s")

        def zb(z, carry):
            @pl.when(lax.rem(z, _NTILES) == s)
            def _():
                pltpu.sync_copy(zrows, acc.at[pl.ds(z * _ZROWS, _ZROWS)])
            return carry
        lax.fori_loop(0, nz // _ZROWS, zb, 0)
        if stage:
            def stage_in(nf):
                def sb(z, carry):
                    @pl.when(lax.rem(z, _NTILES) == s)
                    def _():
                        pltpu.sync_copy(nf.at[pl.ds(z * _ZROWS, _ZROWS)],
                                        table.at[pl.ds(z * _ZROWS, _ZROWS)])
                    return carry
                lax.fori_loop(0, n // _ZROWS, sb, 0)

            @pl.when(c == 0)
            def _():
                stage_in(nf0)

            @pl.when(c == 1)
            def _():
                stage_in(nf1)
        plsc.subcore_barrier()

        def edges(nf):
            buf_bytes_src = nf.at[pl.ds(0, _CH * _LANES)]
            src = table if stage else nf

            def half(k, g0, idxv, bufv, semg, sems):
                @pl.when(k > 0)
                def _():
                    pltpu.make_async_copy(buf_bytes_src, bufv, sems).wait()
                pltpu.sync_copy(idx2.at[pl.ds(g0, _CH)], idxv)
                return [pltpu.async_copy(src.at[idxv.at[j, 0]],
                                         bufv.at[pl.ds(j * _LANES, _LANES)],
                                         semg)
                        for j in range(_CH)]

            def scatter(idxv, bufv, sems, gh):
                for h in gh:
                    h.wait()
                for j in range(_CH):
                    pltpu.async_copy(bufv.at[pl.ds(j * _LANES, _LANES)],
                                     acc.at[idxv.at[j, 1]], sems, add=True)

            def cb(k, carry):
                g0 = s * gpt + k * (2 * _CH)
                gha = half(k, g0, idxa, bufa, semga, semsa)
                ghb = half(k, g0 + _CH, idxb, bufb, semgb, semsb)
                scatter(idxa, bufa, semsa, gha)
                scatter(idxb, bufb, semsb, ghb)
                return carry
            lax.fori_loop(0, npair, cb, 0)
            pltpu.make_async_copy(buf_bytes_src, bufa, semsa).wait()
            pltpu.make_async_copy(buf_bytes_src, bufb, semsb).wait()

        @pl.when(c == 0)
        def _():
            edges(nf0)

        @pl.when(c == 1)
        def _():
            edges(nf1)

        plsc.subcore_barrier()

        def drain(out):
            def db(d, carry):
                @pl.when(lax.rem(d, _NTILES) == s)
                def _():
                    pltpu.sync_copy(acc.at[pl.ds(d * _ZROWS, _ZROWS)],
                                    out.at[pl.ds(d * _ZROWS, _ZROWS)])
                return carry
            lax.fori_loop(0, n // _ZROWS, db, 0)

        @pl.when(c == 0)
        def _():
            drain(out0)

        @pl.when(c == 1)
        def _():
            drain(out1)

    return agg


@functools.lru_cache(maxsize=None)
def _make_sc_agg4(n, nz, g):
    gpt = g // _NTILES
    npair = gpt // (2 * _CH)
    mesh = plsc.VectorSubcoreMesh(core_axis_name="c", subcore_axis_name="s")

    @functools.partial(
        pl.kernel,
        mesh=mesh,
        compiler_params=pltpu.CompilerParams(use_tc_tiling_on_sc=False),
        out_type=[jax.ShapeDtypeStruct((n, 16), _F32)] * 4,
        scratch_types=[
            pltpu.VMEM((_CH, 2, _LANES), jnp.int32),
            pltpu.VMEM((_CH, 2, _LANES), jnp.int32),
            pltpu.VMEM((_CH * _LANES, 16), _F32),
            pltpu.VMEM((_CH * _LANES, 16), _F32),
            pltpu.VMEM_SHARED((nz, 16), _F32),
            pltpu.VMEM_SHARED((n, 16), _F32),
            pltpu.SemaphoreType.DMA,
            pltpu.SemaphoreType.DMA,
            pltpu.SemaphoreType.DMA,
            pltpu.SemaphoreType.DMA,
        ],
    )
    def agg(nf0, nf1, nf2, nf3, idx2, zq, out0, out1, out2, out3,
            idxa, idxb, bufa, bufb, acc, table, semga, semgb, semsa, semsb):
        c = lax.axis_index("c")
        s = lax.axis_index("s")

        def zero_and_stage(nf):
            def zb(z, carry):
                @pl.when(lax.rem(z, _NTILES) == s)
                def _():
                    pltpu.sync_copy(zq, acc.at[pl.ds(z * _ZROWS, _ZROWS)])
                return carry
            lax.fori_loop(0, nz // _ZROWS, zb, 0)

            def sb(z, carry):
                @pl.when(lax.rem(z, _NTILES) == s)
                def _():
                    pltpu.sync_copy(nf.at[pl.ds(z * _ZROWS, _ZROWS)],
                                    table.at[pl.ds(z * _ZROWS, _ZROWS)])
                return carry
            lax.fori_loop(0, n // _ZROWS, sb, 0)

        def edges(nf):
            buf_bytes_src = nf.at[pl.ds(0, _CH * _LANES)]

            def half(k, g0, idxv, bufv, semg, sems):
                @pl.when(k > 0)
                def _():
                    pltpu.make_async_copy(buf_bytes_src, bufv, sems).wait()
                pltpu.sync_copy(idx2.at[pl.ds(g0, _CH)], idxv)
                return [pltpu.async_copy(table.at[idxv.at[j, 0]],
                                         bufv.at[pl.ds(j * _LANES, _LANES)],
                                         semg)
                        for j in range(_CH)]

            def scatter(idxv, bufv, sems, gh):
                for h in gh:
                    h.wait()
                for j in range(_CH):
                    pltpu.async_copy(bufv.at[pl.ds(j * _LANES, _LANES)],
                                     acc.at[idxv.at[j, 1]], sems, add=True)

            def cb(k, carry):
                g0 = s * gpt + k * (2 * _CH)
                gha = half(k, g0, idxa, bufa, semga, semsa)
                ghb = half(k, g0 + _CH, idxb, bufb, semgb, semsb)
                scatter(idxa, bufa, semsa, gha)
                scatter(idxb, bufb, semsb, ghb)
                return carry
            lax.fori_loop(0, npair, cb, 0)
            pltpu.make_async_copy(buf_bytes_src, bufa, semsa).wait()
            pltpu.make_async_copy(buf_bytes_src, bufb, semsb).wait()

        def drain(out):
            def db(d, carry):
                @pl.when(lax.rem(d, _NTILES) == s)
                def _():
                    pltpu.sync_copy(acc.at[pl.ds(d * _ZROWS, _ZROWS)],
                                    out.at[pl.ds(d * _ZROWS, _ZROWS)])
                return carry
            lax.fori_loop(0, n // _ZROWS, db, 0)

        for p, (nf_c0, nf_c1, out_c0, out_c1) in enumerate(
                [(nf0, nf2, out0, out2), (nf1, nf3, out1, out3)]):
            @pl.when(c == 0)
            def _(nf=nf_c0):
                zero_and_stage(nf)

            @pl.when(c == 1)
            def _(nf=nf_c1):
                zero_and_stage(nf)

            plsc.subcore_barrier()

            @pl.when(c == 0)
            def _(nf=nf_c0):
                edges(nf)

            @pl.when(c == 1)
            def _(nf=nf_c1):
                edges(nf)

            plsc.subcore_barrier()

            @pl.when(c == 0)
            def _(out=out_c0):
                drain(out)

            @pl.when(c == 1)
            def _(out=out_c1):
                drain(out)

            if p == 0:
                plsc.subcore_barrier()

    return agg


_BN = 1000


def _mm_split(emb, w):
    n = emb.shape[0]

    def body(e_ref, w_ref, o0_ref, o1_ref):
        p = jnp.dot(e_ref[...], w_ref[...], preferred_element_type=_F32)
        o0_ref[...] = p[:, :32]
        o1_ref[...] = p[:, 32:]

    return pl.pallas_call(
        body,
        grid=(n // _BN,),
        in_specs=[pl.BlockSpec((_BN, 64), lambda i: (i, 0)),
                  pl.BlockSpec((64, 64), lambda i: (0, 0))],
        out_specs=[pl.BlockSpec((_BN, 32), lambda i: (i, 0)),
                   pl.BlockSpec((_BN, 32), lambda i: (i, 0))],
        out_shape=[jax.ShapeDtypeStruct((n, 32), _F32),
                   jax.ShapeDtypeStruct((n, 32), _F32)],
    )(emb, w)


def _mid_layer(a0, a1, w):
    n = a0.shape[0]

    def body(a0_ref, a1_ref, w_ref, on_ref, o0_ref, o1_ref):
        e = jnp.concatenate([_leaky(a0_ref[...]), _leaky(a1_ref[...])], axis=1)
        nrm = jnp.sqrt(jnp.sum(e * e, axis=1, keepdims=True))
        on_ref[...] = e / jnp.maximum(nrm, 1e-12)
        p = jnp.dot(e, w_ref[...], preferred_element_type=_F32)
        o0_ref[...] = p[:, :32]
        o1_ref[...] = p[:, 32:]

    return pl.pallas_call(
        body,
        grid=(n // _BN,),
        in_specs=[pl.BlockSpec((_BN, 32), lambda i: (i, 0)),
                  pl.BlockSpec((_BN, 32), lambda i: (i, 0)),
                  pl.BlockSpec((64, 64), lambda i: (0, 0))],
        out_specs=[pl.BlockSpec((_BN, 64), lambda i: (i, 0)),
                   pl.BlockSpec((_BN, 32), lambda i: (i, 0)),
                   pl.BlockSpec((_BN, 32), lambda i: (i, 0))],
        out_shape=[jax.ShapeDtypeStruct((n, 64), _F32),
                   jax.ShapeDtypeStruct((n, 32), _F32),
                   jax.ShapeDtypeStruct((n, 32), _F32)],
    )(a0, a1, w)


def _final_layer(a0, a1, base, n1):
    n = a0.shape[0]

    def body(a0_ref, a1_ref, b_ref, n1_ref, o_ref):
        e = jnp.concatenate([_leaky(a0_ref[...]), _leaky(a1_ref[...])], axis=1)
        nrm = jnp.sqrt(jnp.sum(e * e, axis=1, keepdims=True))
        o_ref[...] = b_ref[...] + n1_ref[...] + e / jnp.maximum(nrm, 1e-12)

    return pl.pallas_call(
        body,
        grid=(n // _BN,),
        in_specs=[pl.BlockSpec((_BN, 32), lambda i: (i, 0)),
                  pl.BlockSpec((_BN, 32), lambda i: (i, 0)),
                  pl.BlockSpec((_BN, 64), lambda i: (i, 0)),
                  pl.BlockSpec((_BN, 64), lambda i: (i, 0))],
        out_specs=pl.BlockSpec((_BN, 64), lambda i: (i, 0)),
        out_shape=jax.ShapeDtypeStruct((n, 64), _F32),
    )(a0, a1, base, n1)


def _mm_split4(emb, w):
    n = emb.shape[0]

    def body(e_ref, w_ref, o0_ref, o1_ref, o2_ref, o3_ref):
        p = jnp.dot(e_ref[...], w_ref[...], preferred_element_type=_F32)
        o0_ref[...] = p[:, 0:16]
        o1_ref[...] = p[:, 16:32]
        o2_ref[...] = p[:, 32:48]
        o3_ref[...] = p[:, 48:64]

    return pl.pallas_call(
        body,
        grid=(n // _BN,),
        in_specs=[pl.BlockSpec((_BN, 64), lambda i: (i, 0)),
                  pl.BlockSpec((64, 64), lambda i: (0, 0))],
        out_specs=[pl.BlockSpec((_BN, 16), lambda i: (i, 0))] * 4,
        out_shape=[jax.ShapeDtypeStruct((n, 16), _F32)] * 4,
    )(emb, w)


def _mid_layer4(a0, a1, a2, a3, w):
    n = a0.shape[0]

    def body(a0_ref, a1_ref, a2_ref, a3_ref, w_ref,
             on_ref, o0_ref, o1_ref, o2_ref, o3_ref):
        e = jnp.concatenate([_leaky(a0_ref[...]), _leaky(a1_ref[...]),
                             _leaky(a2_ref[...]), _leaky(a3_ref[...])], axis=1)
        nrm = jnp.sqrt(jnp.sum(e * e, axis=1, keepdims=True))
        on_ref[...] = e / jnp.maximum(nrm, 1e-12)
        p = jnp.dot(e, w_ref[...], preferred_element_type=_F32)
        o0_ref[...] = p[:, 0:16]
        o1_ref[...] = p[:, 16:32]
        o2_ref[...] = p[:, 32:48]
        o3_ref[...] = p[:, 48:64]

    return pl.pallas_call(
        body,
        grid=(n // _BN,),
        in_specs=[pl.BlockSpec((_BN, 16), lambda i: (i, 0))] * 4 +
                 [pl.BlockSpec((64, 64), lambda i: (0, 0))],
        out_specs=[pl.BlockSpec((_BN, 64), lambda i: (i, 0))] +
                  [pl.BlockSpec((_BN, 16), lambda i: (i, 0))] * 4,
        out_shape=[jax.ShapeDtypeStruct((n, 64), _F32)] +
                  [jax.ShapeDtypeStruct((n, 16), _F32)] * 4,
    )(a0, a1, a2, a3, w)


def _final_layer4(a0, a1, a2, a3, base, n1):
    n = a0.shape[0]

    def body(a0_ref, a1_ref, a2_ref, a3_ref, b_ref, n1_ref, o_ref):
        e = jnp.concatenate([_leaky(a0_ref[...]), _leaky(a1_ref[...]),
                             _leaky(a2_ref[...]), _leaky(a3_ref[...])], axis=1)
        nrm = jnp.sqrt(jnp.sum(e * e, axis=1, keepdims=True))
        o_ref[...] = b_ref[...] + n1_ref[...] + e / jnp.maximum(nrm, 1e-12)

    return pl.pallas_call(
        body,
        grid=(n // _BN,),
        in_specs=[pl.BlockSpec((_BN, 16), lambda i: (i, 0))] * 4 +
                 [pl.BlockSpec((_BN, 64), lambda i: (i, 0))] * 2,
        out_specs=pl.BlockSpec((_BN, 64), lambda i: (i, 0)),
        out_shape=jax.ShapeDtypeStruct((n, 64), _F32),
    )(a0, a1, a2, a3, base, n1)


def _prep_edges(adj, e_pad, trash):
    row = adj[0].astype(jnp.int32)
    col = adj[1].astype(jnp.int32)
    pad = e_pad - row.shape[0]
    col_p = jnp.concatenate([col, jnp.zeros((pad,), jnp.int32)])
    row_p = jnp.concatenate([row, jnp.full((pad,), trash, jnp.int32)])
    return jnp.stack([col_p.reshape(-1, _LANES),
                      row_p.reshape(-1, _LANES)], axis=1)


def kernel(ui_adj, social_adj, user_emb, item_emb, W_ui0, W_ui1, W_s0, W_s1):
    n_user = user_emb.shape[0]
    n_ui = n_user + item_emb.shape[0]

    e0 = jnp.concatenate([user_emb, item_emb], axis=0)
    zrows = jnp.zeros((_ZROWS, 32), _F32)

    ui_idx = _prep_edges(ui_adj, 811008, n_ui)
    s_idx = _prep_edges(social_adj, 405504, n_user)

    agg_ui = _make_sc_agg4(n_ui, 51000, ui_idx.shape[0])
    agg_s = _make_sc_agg(n_user, 26000, s_idx.shape[0], stage=True)
    zq = jnp.zeros((_ZROWS, 16), _F32)

    nfu = _mm_split4(e0, W_ui0)
    au = agg_ui(*nfu, ui_idx, zq)
    nu1, *nfub = _mid_layer4(*au, W_ui1)
    bu = agg_ui(*nfub, ui_idx, zq)
    ui_emb = _final_layer4(*bu, e0, nu1)

    nfs0, nfs1 = _mm_split(user_emb, W_s0)
    as0, as1 = agg_s(nfs0, nfs1, s_idx, zrows)
    ns1, nfs0b, nfs1b = _mid_layer(as0, as1, W_s1)
    bs0, bs1 = agg_s(nfs0b, nfs1b, s_idx, zrows)
    social_emb = _final_layer(bs0, bs1, user_emb, ns1)

    return (ui_emb, social_emb)


# speedup vs baseline: 1.0865x; 1.0865x over previous
import functools

import jax
import jax.numpy as jnp
from jax import lax
from jax.experimental import pallas as pl
from jax.experimental.pallas import tpu as pltpu
from jax.experimental.pallas import tpu_sc as plsc

_F32 = jnp.float32
_LANES = 384
_CH = 1
_NTILES = 16
_ZROWS = 1000


def _leaky(x):
    return jnp.where(x >= 0, x, 0.5 * x)


@functools.lru_cache(maxsize=None)
def _make_sc_agg(n, nz, g, stage=False):
    gpt = g // _NTILES
    npair = gpt // (2 * _CH)
    mesh = plsc.VectorSubcoreMesh(core_axis_name="c", subcore_axis_name="s")

    @functools.partial(
        pl.kernel,
        mesh=mesh,
        compiler_params=pltpu.CompilerParams(use_tc_tiling_on_sc=False),
        out_type=[jax.ShapeDtypeStruct((n, 32), _F32),
                  jax.ShapeDtypeStruct((n, 32), _F32)],
        scratch_types=[
            pltpu.VMEM((_CH, 2, _LANES), jnp.int32),
            pltpu.VMEM((_CH, 2, _LANES), jnp.int32),
            pltpu.VMEM((_CH * _LANES, 32), _F32),
            pltpu.VMEM((_CH * _LANES, 32), _F32),
            pltpu.VMEM_SHARED((nz, 32), _F32),
        ] + ([pltpu.VMEM_SHARED((n, 32), _F32)] if stage else []) + [
            pltpu.SemaphoreType.DMA,
            pltpu.SemaphoreType.DMA,
            pltpu.SemaphoreType.DMA,
            pltpu.SemaphoreType.DMA,
        ],
    )
    def agg(nf0, nf1, idx2, zrows, out0, out1, *refs):
        if stage:
            idxa, idxb, bufa, bufb, acc, table, semga, semgb, semsa, semsb = refs
        else:
            idxa, idxb, bufa, bufb, acc, semga, semgb, semsa, semsb = refs
            table = None
        c = lax.axis_index("c")
        s = lax.axis_index("s")

        def zb(z, carry):
            @pl.when(lax.rem(z, _NTILES) == s)
            def _():
                pltpu.sync_copy(zrows, acc.at[pl.ds(z * _ZROWS, _ZROWS)])
            return carry
        lax.fori_loop(0, nz // _ZROWS, zb, 0)
        if stage:
            def stage_in(nf):
                def sb(z, carry):
                    @pl.when(lax.rem(z, _NTILES) == s)
                    def _():
                        pltpu.sync_copy(nf.at[pl.ds(z * _ZROWS, _ZROWS)],
                                        table.at[pl.ds(z * _ZROWS, _ZROWS)])
                    return carry
                lax.fori_loop(0, n // _ZROWS, sb, 0)

            @pl.when(c == 0)
            def _():
                stage_in(nf0)

            @pl.when(c == 1)
            def _():
                stage_in(nf1)
        plsc.subcore_barrier()

        def edges(nf):
            buf_bytes_src = nf.at[pl.ds(0, _CH * _LANES)]
            src = table if stage else nf

            def half(k, g0, idxv, bufv, semg, sems):
                @pl.when(k > 0)
                def _():
                    pltpu.make_async_copy(buf_bytes_src, bufv, sems).wait()
                pltpu.sync_copy(idx2.at[pl.ds(g0, _CH)], idxv)
                return [pltpu.async_copy(src.at[idxv.at[j, 0]],
                                         bufv.at[pl.ds(j * _LANES, _LANES)],
                                         semg)
                        for j in range(_CH)]

            def scatter(idxv, bufv, sems, gh):
                for h in gh:
                    h.wait()
                for j in range(_CH):
                    pltpu.async_copy(bufv.at[pl.ds(j * _LANES, _LANES)],
                                     acc.at[idxv.at[j, 1]], sems, add=True)

            def cb(k, carry):
                g0 = s * gpt + k * (2 * _CH)
                gha = half(k, g0, idxa, bufa, semga, semsa)
                ghb = half(k, g0 + _CH, idxb, bufb, semgb, semsb)
                scatter(idxa, bufa, semsa, gha)
                scatter(idxb, bufb, semsb, ghb)
                return carry
            lax.fori_loop(0, npair, cb, 0)
            pltpu.make_async_copy(buf_bytes_src, bufa, semsa).wait()
            pltpu.make_async_copy(buf_bytes_src, bufb, semsb).wait()

        @pl.when(c == 0)
        def _():
            edges(nf0)

        @pl.when(c == 1)
        def _():
            edges(nf1)

        plsc.subcore_barrier()

        def drain(out):
            def db(d, carry):
                @pl.when(lax.rem(d, _NTILES) == s)
                def _():
                    pltpu.sync_copy(acc.at[pl.ds(d * _ZROWS, _ZROWS)],
                                    out.at[pl.ds(d * _ZROWS, _ZROWS)])
                return carry
            lax.fori_loop(0, n // _ZROWS, db, 0)

        @pl.when(c == 0)
        def _():
            drain(out0)

        @pl.when(c == 1)
        def _():
            drain(out1)

    return agg


@functools.lru_cache(maxsize=None)
def _make_sc_agg4(n, nz, g):
    gpt = g // _NTILES
    npair = gpt // (2 * _CH)
    mesh = plsc.VectorSubcoreMesh(core_axis_name="c", subcore_axis_name="s")

    @functools.partial(
        pl.kernel,
        mesh=mesh,
        compiler_params=pltpu.CompilerParams(use_tc_tiling_on_sc=False),
        out_type=[jax.ShapeDtypeStruct((n, 16), _F32)] * 4,
        scratch_types=[
            pltpu.VMEM((_CH, 2, _LANES), jnp.int32),
            pltpu.VMEM((_CH, 2, _LANES), jnp.int32),
            pltpu.VMEM((_CH * _LANES, 16), _F32),
            pltpu.VMEM((_CH * _LANES, 16), _F32),
            pltpu.VMEM_SHARED((nz, 16), _F32),
            pltpu.VMEM_SHARED((n, 16), _F32),
            pltpu.SemaphoreType.DMA,
            pltpu.SemaphoreType.DMA,
            pltpu.SemaphoreType.DMA,
            pltpu.SemaphoreType.DMA,
        ],
    )
    def agg(nf0, nf1, nf2, nf3, idx2, zq, out0, out1, out2, out3,
            idxa, idxb, bufa, bufb, acc, table, semga, semgb, semsa, semsb):
        c = lax.axis_index("c")
        s = lax.axis_index("s")

        def zero_and_stage(nf):
            def zb(z, carry):
                @pl.when(lax.rem(z, _NTILES) == s)
                def _():
                    pltpu.sync_copy(zq, acc.at[pl.ds(z * _ZROWS, _ZROWS)])
                return carry
            lax.fori_loop(0, nz // _ZROWS, zb, 0)

            def sb(z, carry):
                @pl.when(lax.rem(z, _NTILES) == s)
                def _():
                    pltpu.sync_copy(nf.at[pl.ds(z * _ZROWS, _ZROWS)],
                                    table.at[pl.ds(z * _ZROWS, _ZROWS)])
                return carry
            lax.fori_loop(0, n // _ZROWS, sb, 0)

        def edges(nf):
            buf_bytes_src = nf.at[pl.ds(0, _CH * _LANES)]

            def half(k, g0, idxv, bufv, semg, sems):
                @pl.when(k > 0)
                def _():
                    pltpu.make_async_copy(buf_bytes_src, bufv, sems).wait()
                pltpu.sync_copy(idx2.at[pl.ds(g0, _CH)], idxv)
                return [pltpu.async_copy(table.at[idxv.at[j, 0]],
                                         bufv.at[pl.ds(j * _LANES, _LANES)],
                                         semg)
                        for j in range(_CH)]

            def scatter(idxv, bufv, sems, gh):
                for h in gh:
                    h.wait()
                for j in range(_CH):
                    pltpu.async_copy(bufv.at[pl.ds(j * _LANES, _LANES)],
                                     acc.at[idxv.at[j, 1]], sems, add=True)

            def cb(k, carry):
                g0 = s * gpt + k * (2 * _CH)
                gha = half(k, g0, idxa, bufa, semga, semsa)
                ghb = half(k, g0 + _CH, idxb, bufb, semgb, semsb)
                scatter(idxa, bufa, semsa, gha)
                scatter(idxb, bufb, semsb, ghb)
                return carry
            lax.fori_loop(0, npair, cb, 0)
            pltpu.make_async_copy(buf_bytes_src, bufa, semsa).wait()
            pltpu.make_async_copy(buf_bytes_src, bufb, semsb).wait()

        def drain(out):
            def db(d, carry):
                @pl.when(lax.rem(d, _NTILES) == s)
                def _():
                    pltpu.sync_copy(acc.at[pl.ds(d * _ZROWS, _ZROWS)],
                                    out.at[pl.ds(d * _ZROWS, _ZROWS)])
                return carry
            lax.fori_loop(0, n // _ZROWS, db, 0)

        for p, (nf_c0, nf_c1, out_c0, out_c1) in enumerate(
                [(nf0, nf2, out0, out2), (nf1, nf3, out1, out3)]):
            @pl.when(c == 0)
            def _(nf=nf_c0):
                zero_and_stage(nf)

            @pl.when(c == 1)
            def _(nf=nf_c1):
                zero_and_stage(nf)

            plsc.subcore_barrier()

            @pl.when(c == 0)
            def _(nf=nf_c0):
                edges(nf)

            @pl.when(c == 1)
            def _(nf=nf_c1):
                edges(nf)

            plsc.subcore_barrier()

            @pl.when(c == 0)
            def _(out=out_c0):
                drain(out)

            @pl.when(c == 1)
            def _(out=out_c1):
                drain(out)

            if p == 0:
                plsc.subcore_barrier()

    return agg


_BN = 1000


def _mm_split(emb, w):
    n = emb.shape[0]

    def body(e_ref, w_ref, o0_ref, o1_ref):
        p = jnp.dot(e_ref[...], w_ref[...], preferred_element_type=_F32)
        o0_ref[...] = p[:, :32]
        o1_ref[...] = p[:, 32:]

    return pl.pallas_call(
        body,
        grid=(n // _BN,),
        in_specs=[pl.BlockSpec((_BN, 64), lambda i: (i, 0)),
                  pl.BlockSpec((64, 64), lambda i: (0, 0))],
        out_specs=[pl.BlockSpec((_BN, 32), lambda i: (i, 0)),
                   pl.BlockSpec((_BN, 32), lambda i: (i, 0))],
        out_shape=[jax.ShapeDtypeStruct((n, 32), _F32),
                   jax.ShapeDtypeStruct((n, 32), _F32)],
    )(emb, w)


def _mid_layer(a0, a1, w):
    n = a0.shape[0]

    def body(a0_ref, a1_ref, w_ref, on_ref, o0_ref, o1_ref):
        e = jnp.concatenate([_leaky(a0_ref[...]), _leaky(a1_ref[...])], axis=1)
        nrm = jnp.sqrt(jnp.sum(e * e, axis=1, keepdims=True))
        on_ref[...] = e / jnp.maximum(nrm, 1e-12)
        p = jnp.dot(e, w_ref[...], preferred_element_type=_F32)
        o0_ref[...] = p[:, :32]
        o1_ref[...] = p[:, 32:]

    return pl.pallas_call(
        body,
        grid=(n // _BN,),
        in_specs=[pl.BlockSpec((_BN, 32), lambda i: (i, 0)),
                  pl.BlockSpec((_BN, 32), lambda i: (i, 0)),
                  pl.BlockSpec((64, 64), lambda i: (0, 0))],
        out_specs=[pl.BlockSpec((_BN, 64), lambda i: (i, 0)),
                   pl.BlockSpec((_BN, 32), lambda i: (i, 0)),
                   pl.BlockSpec((_BN, 32), lambda i: (i, 0))],
        out_shape=[jax.ShapeDtypeStruct((n, 64), _F32),
                   jax.ShapeDtypeStruct((n, 32), _F32),
                   jax.ShapeDtypeStruct((n, 32), _F32)],
    )(a0, a1, w)


def _final_layer(a0, a1, base, n1):
    n = a0.shape[0]

    def body(a0_ref, a1_ref, b_ref, n1_ref, o_ref):
        e = jnp.concatenate([_leaky(a0_ref[...]), _leaky(a1_ref[...])], axis=1)
        nrm = jnp.sqrt(jnp.sum(e * e, axis=1, keepdims=True))
        o_ref[...] = b_ref[...] + n1_ref[...] + e / jnp.maximum(nrm, 1e-12)

    return pl.pallas_call(
        body,
        grid=(n // _BN,),
        in_specs=[pl.BlockSpec((_BN, 32), lambda i: (i, 0)),
                  pl.BlockSpec((_BN, 32), lambda i: (i, 0)),
                  pl.BlockSpec((_BN, 64), lambda i: (i, 0)),
                  pl.BlockSpec((_BN, 64), lambda i: (i, 0))],
        out_specs=pl.BlockSpec((_BN, 64), lambda i: (i, 0)),
        out_shape=jax.ShapeDtypeStruct((n, 64), _F32),
    )(a0, a1, base, n1)


def _mm_split4(emb, w):
    n = emb.shape[0]

    def body(e_ref, w_ref, o0_ref, o1_ref, o2_ref, o3_ref):
        p = jnp.dot(e_ref[...], w_ref[...], preferred_element_type=_F32)
        o0_ref[...] = p[:, 0:16]
        o1_ref[...] = p[:, 16:32]
        o2_ref[...] = p[:, 32:48]
        o3_ref[...] = p[:, 48:64]

    return pl.pallas_call(
        body,
        grid=(n // _BN,),
        in_specs=[pl.BlockSpec((_BN, 64), lambda i: (i, 0)),
                  pl.BlockSpec((64, 64), lambda i: (0, 0))],
        out_specs=[pl.BlockSpec((_BN, 16), lambda i: (i, 0))] * 4,
        out_shape=[jax.ShapeDtypeStruct((n, 16), _F32)] * 4,
    )(emb, w)


def _mid_layer4(a0, a1, a2, a3, w):
    n = a0.shape[0]

    def body(a0_ref, a1_ref, a2_ref, a3_ref, w_ref,
             on_ref, o0_ref, o1_ref, o2_ref, o3_ref):
        e = jnp.concatenate([_leaky(a0_ref[...]), _leaky(a1_ref[...]),
                             _leaky(a2_ref[...]), _leaky(a3_ref[...])], axis=1)
        nrm = jnp.sqrt(jnp.sum(e * e, axis=1, keepdims=True))
        on_ref[...] = e / jnp.maximum(nrm, 1e-12)
        p = jnp.dot(e, w_ref[...], preferred_element_type=_F32)
        o0_ref[...] = p[:, 0:16]
        o1_ref[...] = p[:, 16:32]
        o2_ref[...] = p[:, 32:48]
        o3_ref[...] = p[:, 48:64]

    return pl.pallas_call(
        body,
        grid=(n // _BN,),
        in_specs=[pl.BlockSpec((_BN, 16), lambda i: (i, 0))] * 4 +
                 [pl.BlockSpec((64, 64), lambda i: (0, 0))],
        out_specs=[pl.BlockSpec((_BN, 64), lambda i: (i, 0))] +
                  [pl.BlockSpec((_BN, 16), lambda i: (i, 0))] * 4,
        out_shape=[jax.ShapeDtypeStruct((n, 64), _F32)] +
                  [jax.ShapeDtypeStruct((n, 16), _F32)] * 4,
    )(a0, a1, a2, a3, w)


def _final_layer4(a0, a1, a2, a3, base, n1):
    n = a0.shape[0]

    def body(a0_ref, a1_ref, a2_ref, a3_ref, b_ref, n1_ref, o_ref):
        e = jnp.concatenate([_leaky(a0_ref[...]), _leaky(a1_ref[...]),
                             _leaky(a2_ref[...]), _leaky(a3_ref[...])], axis=1)
        nrm = jnp.sqrt(jnp.sum(e * e, axis=1, keepdims=True))
        o_ref[...] = b_ref[...] + n1_ref[...] + e / jnp.maximum(nrm, 1e-12)

    return pl.pallas_call(
        body,
        grid=(n // _BN,),
        in_specs=[pl.BlockSpec((_BN, 16), lambda i: (i, 0))] * 4 +
                 [pl.BlockSpec((_BN, 64), lambda i: (i, 0))] * 2,
        out_specs=pl.BlockSpec((_BN, 64), lambda i: (i, 0)),
        out_shape=jax.ShapeDtypeStruct((n, 64), _F32),
    )(a0, a1, a2, a3, base, n1)


def _prep_edges(adj, e_pad, trash):
    row = adj[0].astype(jnp.int32)
    col = adj[1].astype(jnp.int32)
    pad = e_pad - row.shape[0]
    col_p = jnp.concatenate([col, jnp.zeros((pad,), jnp.int32)])
    row_p = jnp.concatenate([row, jnp.full((pad,), trash, jnp.int32)])
    return jnp.stack([col_p.reshape(-1, _LANES),
                      row_p.reshape(-1, _LANES)], axis=1)


def kernel(ui_adj, social_adj, user_emb, item_emb, W_ui0, W_ui1, W_s0, W_s1):
    n_user = user_emb.shape[0]
    n_ui = n_user + item_emb.shape[0]

    e0 = jnp.concatenate([user_emb, item_emb], axis=0)
    zrows = jnp.zeros((_ZROWS, 32), _F32)

    ui_idx = _prep_edges(ui_adj, 811008, n_ui)
    s_idx = _prep_edges(social_adj, 405504, n_user)

    agg_ui = _make_sc_agg(n_ui, 51000, ui_idx.shape[0])
    agg_s = _make_sc_agg(n_user, 26000, s_idx.shape[0], stage=True)

    nfu0, nfu1 = _mm_split(e0, W_ui0)
    au0, au1 = agg_ui(nfu0, nfu1, ui_idx, zrows)
    nu1, nfu0b, nfu1b = _mid_layer(au0, au1, W_ui1)
    bu0, bu1 = agg_ui(nfu0b, nfu1b, ui_idx, zrows)
    ui_emb = _final_layer(bu0, bu1, e0, nu1)

    nfs0, nfs1 = _mm_split(user_emb, W_s0)
    as0, as1 = agg_s(nfs0, nfs1, s_idx, zrows)
    ns1, nfs0b, nfs1b = _mid_layer(as0, as1, W_s1)
    bs0, bs1 = agg_s(nfs0b, nfs1b, s_idx, zrows)
    social_emb = _final_layer(bs0, bs1, user_emb, ns1)

    return (ui_emb, social_emb)
